# Initial kernel scaffold; baseline (speedup 1.0000x reference)
import jax
import jax.numpy as jnp
from jax.experimental import pallas as pl


def kernel(token, week, hour, duration, token_table, week_table, hour_table, duration_table):
    raise NotImplementedError("write your pallas kernel here")


# speedup vs baseline: 5.5008x; 5.5008x over previous
import jax
import jax.numpy as jnp
from jax import lax
from jax.experimental import pallas as pl
from jax.experimental.pallas import tpu as pltpu
from jax.experimental.pallas import tpu_sc as plsc

_B, _L = 4096, 200
_N = _B * _L
_TOK_D = 64
_SMALL_D = 48
_OUT_D = _TOK_D + _SMALL_D
_NC, _NS = 2, 16
_NW = _NC * _NS
_ROWS_W = _N // _NW
_CG = 128
_NCH = _ROWS_W // _CG
_NBUF = 4
_BLK = 3200
_NBLK = _ROWS_W // _BLK


def _body(tok_hbm, wk_hbm, hr_hbm, du_hbm, tok_tab, comb_tab, out_hbm,
          tok_idx, cidx, wbuf, hbuf, dbuf, tok_rows, small_rows,
          sem_tok, sem_idx, sem_t, sem_s, sem_w):
    wid = lax.axis_index("c") * _NS + lax.axis_index("s")
    rbase = pl.multiple_of(wid * _ROWS_W, _ROWS_W)

    tok_cp = pltpu.async_copy(tok_hbm.at[pl.ds(rbase, _ROWS_W)], tok_idx,
                              sem_tok)
    for blk in range(_NBLK):
        off = rbase + blk * _BLK
        cw = pltpu.async_copy(wk_hbm.at[pl.ds(off, _BLK)], wbuf, sem_idx)
        ch = pltpu.async_copy(hr_hbm.at[pl.ds(off, _BLK)], hbuf, sem_idx)
        cd = pltpu.async_copy(du_hbm.at[pl.ds(off, _BLK)], dbuf, sem_idx)
        cw.wait()
        ch.wait()
        cd.wait()

        def fuse(j, _):
            s = pl.ds(pl.multiple_of(j * 16, 16), 16)
            w = wbuf[s]
            h = hbuf[s]
            d = dbuf[s]
            so = pl.ds(pl.multiple_of(blk * _BLK + j * 16, 16), 16)
            cidx[so] = (w * 24 + h) * 24 + d
            return _

        lax.fori_loop(0, _BLK // 16, fuse, 0)
    tok_cp.wait()

    def fire(g, slot):
        s = pl.ds(pl.multiple_of(g * _CG, _CG), _CG)
        pltpu.async_copy(tok_tab.at[tok_idx.at[s]], tok_rows.at[slot],
                         sem_t.at[slot])
        pltpu.async_copy(comb_tab.at[cidx.at[s]], small_rows.at[slot],
                         sem_s.at[slot])

    def drain(g, slot):
        s = pl.ds(pl.multiple_of(g * _CG, _CG), _CG)
        pltpu.make_async_copy(tok_tab.at[tok_idx.at[s]], tok_rows.at[slot],
                              sem_t.at[slot]).wait()
        pltpu.make_async_copy(comb_tab.at[cidx.at[s]], small_rows.at[slot],
                              sem_s.at[slot]).wait()

    def put(g, slot):
        r = pl.ds(pl.multiple_of(rbase + g * _CG, _CG), _CG)
        pltpu.async_copy(tok_rows.at[slot],
                         out_hbm.at[r, pl.ds(0, _TOK_D)], sem_w.at[slot])
        pltpu.async_copy(small_rows.at[slot],
                         out_hbm.at[r, pl.ds(_TOK_D, _SMALL_D)],
                         sem_w.at[slot])

    def wait_put(g, slot):
        r = pl.ds(pl.multiple_of(rbase + g * _CG, _CG), _CG)
        pltpu.make_async_copy(tok_rows.at[slot],
                              out_hbm.at[r, pl.ds(0, _TOK_D)],
                              sem_w.at[slot]).wait()
        pltpu.make_async_copy(small_rows.at[slot],
                              out_hbm.at[r, pl.ds(_TOK_D, _SMALL_D)],
                              sem_w.at[slot]).wait()

    for g in range(_NBUF - 1):
        fire(g, g)

    def step(i, _):
        for b in range(_NBUF):
            g = i * _NBUF + b
            b3 = (b + _NBUF - 1) % _NBUF

            @pl.when(g >= 1)
            def _wp():
                wait_put(g - 1, b3)

            @pl.when(g + _NBUF - 1 < _NCH)
            def _f():
                fire(g + _NBUF - 1, b3)

            drain(g, b)
            put(g, b)
        return _

    lax.fori_loop(0, _NCH // _NBUF, step, 0)
    wait_put(_NCH - 1, (_NCH - 1) % _NBUF)


_call = pl.kernel(
    _body,
    out_type=jax.ShapeDtypeStruct((_N, _OUT_D), jnp.float32),
    mesh=plsc.VectorSubcoreMesh(core_axis_name="c", subcore_axis_name="s"),
    compiler_params=pltpu.CompilerParams(use_tc_tiling_on_sc=False),
    scratch_types=[
        pltpu.VMEM((_ROWS_W,), jnp.int32),
        pltpu.VMEM((_ROWS_W,), jnp.int32),
        pltpu.VMEM((_BLK,), jnp.int32),
        pltpu.VMEM((_BLK,), jnp.int32),
        pltpu.VMEM((_BLK,), jnp.int32),
        pltpu.VMEM((_NBUF, _CG, _TOK_D), jnp.float32),
        pltpu.VMEM((_NBUF, _CG, _SMALL_D), jnp.float32),
        pltpu.SemaphoreType.DMA,
        pltpu.SemaphoreType.DMA,
        pltpu.SemaphoreType.DMA((_NBUF,)),
        pltpu.SemaphoreType.DMA((_NBUF,)),
        pltpu.SemaphoreType.DMA((_NBUF,)),
    ],
)


def kernel(token, week, hour, duration, token_table, week_table, hour_table,
           duration_table):
    comb = jnp.concatenate([
        jnp.broadcast_to(week_table[:, None, None, :], (7, 24, 24, 16)),
        jnp.broadcast_to(hour_table[None, :, None, :], (7, 24, 24, 16)),
        jnp.broadcast_to(duration_table[None, None, :, :], (7, 24, 24, 16)),
    ], axis=-1).reshape(7 * 24 * 24, _SMALL_D)
    out = _call(token.reshape(_N), week.reshape(_N), hour.reshape(_N),
                duration.reshape(_N), token_table, comb)
    return out.reshape(_B, _L, _OUT_D)
